# Initial kernel scaffold; baseline (speedup 1.0000x reference)
import jax
import jax.numpy as jnp
from jax.experimental import pallas as pl


def kernel(x, means):
    raise NotImplementedError("write your pallas kernel here")


# speedup vs baseline: 1.2484x; 1.2484x over previous
import functools

import jax
import jax.numpy as jnp
from jax.experimental import pallas as pl
from jax.experimental.pallas import tpu as pltpu

NS = 1024
D = 64
N_CLASSES = 100000

BC_ARG = 2048
BC_OH = 4096
NB_ARG = (N_CLASSES + BC_ARG - 1) // BC_ARG
NB_OH = (N_CLASSES + BC_OH - 1) // BC_OH
BIG_IDX = 2 ** 30


def _argmin_body(x_ref, m_ref, pred_ref, best_val, best_idx):
    b = pl.program_id(0)

    @pl.when(b == 0)
    def _init():
        best_val[...] = jnp.full_like(best_val, jnp.inf)
        best_idx[...] = jnp.zeros_like(best_idx)

    score = jax.lax.dot_general(
        x_ref[...], m_ref[...],
        (((1,), (1,)), ((), ())),
        preferred_element_type=jnp.float32,
    )
    dist = -score
    ids = jax.lax.broadcasted_iota(jnp.int32, (1, BC_ARG), 1) + b * BC_ARG
    dist = jnp.where(ids < N_CLASSES, dist, jnp.inf)
    lmin = jnp.min(dist, axis=1, keepdims=True)
    larg = jnp.min(jnp.where(dist == lmin, ids, BIG_IDX), axis=1,
                   keepdims=True)
    upd = lmin < best_val[...]
    best_idx[...] = jnp.where(upd, larg, best_idx[...])
    best_val[...] = jnp.where(upd, lmin, best_val[...])

    @pl.when(b == NB_ARG - 1)
    def _emit():
        pred_ref[...] = best_idx[...]


def _onehot_body(pred_ref, out_ref):
    b = pl.program_id(0)
    ids = jax.lax.broadcasted_iota(jnp.int32, (1, BC_OH), 1) + b * BC_OH
    out_ref[...] = jnp.where(pred_ref[...] == ids, 1.0, 0.0).astype(jnp.float32)


@functools.partial(jax.jit, static_argnames=())
def kernel(x, means):
    pred = pl.pallas_call(
        _argmin_body,
        grid=(NB_ARG,),
        in_specs=[
            pl.BlockSpec((NS, D), lambda b: (0, 0)),
            pl.BlockSpec((BC_ARG, D), lambda b: (b, 0)),
        ],
        out_specs=pl.BlockSpec((NS, 1), lambda b: (0, 0)),
        out_shape=jax.ShapeDtypeStruct((NS, 1), jnp.int32),
        scratch_shapes=[
            pltpu.VMEM((NS, 1), jnp.float32),
            pltpu.VMEM((NS, 1), jnp.int32),
        ],
        compiler_params=pltpu.CompilerParams(
            dimension_semantics=("arbitrary",),
        ),
    )(x, means)

    out = pl.pallas_call(
        _onehot_body,
        grid=(NB_OH,),
        in_specs=[pl.BlockSpec((NS, 1), lambda b: (0, 0))],
        out_specs=pl.BlockSpec((NS, BC_OH), lambda b: (0, b)),
        out_shape=jax.ShapeDtypeStruct((NS, N_CLASSES), jnp.float32),
        compiler_params=pltpu.CompilerParams(
            dimension_semantics=("arbitrary",),
        ),
    )(pred)
    return out
